# Initial kernel scaffold; baseline (speedup 1.0000x reference)
import jax
import jax.numpy as jnp
from jax.experimental import pallas as pl


def kernel(x, edge_index, W1, b1, W2, b2):
    raise NotImplementedError("write your pallas kernel here")


# speedup vs baseline: 12.9608x; 12.9608x over previous
import functools

import jax
import jax.numpy as jnp
from jax import lax
from jax.experimental import pallas as pl
from jax.experimental.pallas import tpu as pltpu
from jax.experimental.pallas import tpu_sc as plsc

NC = 2
NS = 16
NW = NC * NS
K = 128
LANES = 16


def _mesh():
  return plsc.VectorSubcoreMesh(core_axis_name="c", subcore_axis_name="s")


def _fill_rows(buf, nrows, ncols, value):
  v = jnp.full((LANES,), value, jnp.float32)

  def row(i, carry):
    for l in range(ncols // LANES):
      buf[i, pl.ds(l * LANES, LANES)] = v
    return carry

  lax.fori_loop(0, nrows, row, 0)


def _acc_rows(n):
  blk = NS * K
  return ((n + 1 + blk - 1) // blk) * blk


DEGW = 128


def _make_deg(n, nch):
  acc_rows = _acc_rows(n)
  rpt = acc_rows // NS

  def body(dstv_hbm, out_hbm, dst_v, ones_v, acc):
    cid = lax.axis_index("c")
    sid = lax.axis_index("s")
    wid = sid * NC + cid
    pltpu.sync_copy(dstv_hbm.at[wid], dst_v)
    _fill_rows(ones_v, K, DEGW, 0.0)
    for i in range(rpt // K):
      pltpu.sync_copy(ones_v, acc.at[pl.ds(sid * rpt + i * K, K)])
    _fill_rows(ones_v, K, DEGW, 1.0)
    plsc.subcore_barrier()

    def chunk(j, carry):
      pltpu.sync_copy(ones_v, acc.at[dst_v.at[j]], add=True)
      return carry

    lax.fori_loop(0, nch, chunk, 0)
    plsc.subcore_barrier()
    pltpu.sync_copy(acc.at[pl.ds(sid * rpt, rpt)],
                    out_hbm.at[cid, pl.ds(sid * rpt, rpt)])

  return pl.kernel(
      body,
      out_type=jax.ShapeDtypeStruct((NC, acc_rows, DEGW), jnp.float32),
      mesh=_mesh(),
      scratch_types=[
          pltpu.VMEM((nch, K), jnp.int32),
          pltpu.VMEM((K, DEGW), jnp.float32),
          pltpu.VMEM_SHARED((acc_rows, DEGW), jnp.float32),
      ],
  )


def _make_agg(n, d, nch):
  acc_rows = _acc_rows(n)
  rpt = acc_rows // NS

  def body(xs_hbm, srcv_hbm, dstv_hbm, out_hbm, src_v, dst_v, rows_v, sem,
           acc):
    cid = lax.axis_index("c")
    sid = lax.axis_index("s")
    wid = sid * NC + cid
    pltpu.sync_copy(srcv_hbm.at[wid], src_v)
    pltpu.sync_copy(dstv_hbm.at[wid], dst_v)
    _fill_rows(rows_v, K, d, 0.0)
    for i in range(rpt // K):
      pltpu.sync_copy(rows_v, acc.at[pl.ds(sid * rpt + i * K, K)])
    plsc.subcore_barrier()

    def chunk(j, carry):
      pltpu.async_copy(xs_hbm.at[src_v.at[j]], rows_v, sem).wait()
      pltpu.sync_copy(rows_v, acc.at[dst_v.at[j]], add=True)
      return carry

    lax.fori_loop(0, nch, chunk, 0)
    plsc.subcore_barrier()
    pltpu.sync_copy(acc.at[pl.ds(sid * rpt, rpt)],
                    out_hbm.at[cid, pl.ds(sid * rpt, rpt)])

  return pl.kernel(
      body,
      out_type=jax.ShapeDtypeStruct((NC, acc_rows, d), jnp.float32),
      mesh=_mesh(),
      scratch_types=[
          pltpu.VMEM((nch, K), jnp.int32),
          pltpu.VMEM((nch, K), jnp.int32),
          pltpu.VMEM((K, d), jnp.float32),
          pltpu.SemaphoreType.DMA,
          pltpu.VMEM_SHARED((acc_rows, d), jnp.float32),
      ],
  )


BN = 512


def _prep_body(x_ref, w_ref, degp_ref, xs_ref, dinv_ref):
  deg = degp_ref[0, :, 0:1] + degp_ref[1, :, 0:1] + 1.0
  dinv = lax.rsqrt(deg)
  xw = jnp.dot(x_ref[...], w_ref[...], preferred_element_type=jnp.float32)
  xs_ref[...] = dinv * xw
  dinv_ref[...] = dinv


def _mid_body(p_ref, xs_ref, dinv_ref, b1_ref, z_ref, u_ref):
  dinv = dinv_ref[...]
  agg = p_ref[0] + p_ref[1] + xs_ref[...]
  z = jnp.maximum(dinv * agg + b1_ref[...], 0.0)
  z_ref[...] = z
  u_ref[...] = dinv * z


def _fin_body(q_ref, u_ref, dinv_ref, w2_ref, b2_ref, out_ref):
  acc = q_ref[0] + q_ref[1] + u_ref[...]
  out_ref[...] = (dinv_ref[...]
                  * jnp.dot(acc, w2_ref[...],
                            preferred_element_type=jnp.float32)
                  + b2_ref[...])


def kernel(x, edge_index, W1, b1, W2, b2):
  n, in_dim = x.shape
  hid = W1.shape[1]
  out_dim = W2.shape[1]
  e = edge_index.shape[1]

  nch = -(-e // (NW * K))
  e_pad = NW * nch * K
  src = edge_index[0].astype(jnp.int32)
  dst = edge_index[1].astype(jnp.int32)
  pad = e_pad - e
  srcv = jnp.concatenate([src, jnp.zeros((pad,), jnp.int32)]).reshape(
      NW, nch, K)
  dstv = jnp.concatenate([dst, jnp.full((pad,), n, jnp.int32)]).reshape(
      NW, nch, K)

  degp = _make_deg(n, nch)(dstv)

  grid = (pl.cdiv(n, BN),)
  xs, dinv = pl.pallas_call(
      _prep_body,
      grid=grid,
      in_specs=[
          pl.BlockSpec((BN, in_dim), lambda i: (i, 0)),
          pl.BlockSpec((in_dim, hid), lambda i: (0, 0)),
          pl.BlockSpec((NC, BN, DEGW), lambda i: (0, i, 0)),
      ],
      out_specs=[
          pl.BlockSpec((BN, hid), lambda i: (i, 0)),
          pl.BlockSpec((BN, 1), lambda i: (i, 0)),
      ],
      out_shape=[
          jax.ShapeDtypeStruct((n, hid), jnp.float32),
          jax.ShapeDtypeStruct((n, 1), jnp.float32),
      ],
  )(x, W1, degp)

  agg = _make_agg(n, hid, nch)
  p = agg(xs, srcv, dstv)

  z, u = pl.pallas_call(
      _mid_body,
      grid=grid,
      in_specs=[
          pl.BlockSpec((NC, BN, hid), lambda i: (0, i, 0)),
          pl.BlockSpec((BN, hid), lambda i: (i, 0)),
          pl.BlockSpec((BN, 1), lambda i: (i, 0)),
          pl.BlockSpec((1, hid), lambda i: (0, 0)),
      ],
      out_specs=[
          pl.BlockSpec((BN, hid), lambda i: (i, 0)),
          pl.BlockSpec((BN, hid), lambda i: (i, 0)),
      ],
      out_shape=[
          jax.ShapeDtypeStruct((n, hid), jnp.float32),
          jax.ShapeDtypeStruct((n, hid), jnp.float32),
      ],
  )(p, xs, dinv, b1.reshape(1, hid))

  q = agg(u, srcv, dstv)

  logits = pl.pallas_call(
      _fin_body,
      grid=grid,
      in_specs=[
          pl.BlockSpec((NC, BN, hid), lambda i: (0, i, 0)),
          pl.BlockSpec((BN, hid), lambda i: (i, 0)),
          pl.BlockSpec((BN, 1), lambda i: (i, 0)),
          pl.BlockSpec((hid, out_dim), lambda i: (0, 0)),
          pl.BlockSpec((1, out_dim), lambda i: (0, 0)),
      ],
      out_specs=pl.BlockSpec((BN, out_dim), lambda i: (i, 0)),
      out_shape=jax.ShapeDtypeStruct((n, out_dim), jnp.float32),
  )(q, u, dinv, W2, b2.reshape(1, out_dim))

  return logits, z
